# baseline (speedup 1.0000x reference)
import functools

import jax
import jax.numpy as jnp
from jax import lax
from jax.experimental import pallas as pl
from jax.experimental.pallas import tpu as pltpu
from jax.experimental.pallas import tpu_sc as plsc

VOCAB = 1000000
DIM = 64
B = 4096
L = 200

NUM_CORES = 2
NUM_SUBCORES = 16
NW = NUM_CORES * NUM_SUBCORES
B_PER_W = B // NW
HALF = L // 2
LANES = 16
NCH = DIM // LANES

_mesh = plsc.VectorSubcoreMesh(
    core_axis_name="c", subcore_axis_name="s",
    num_cores=NUM_CORES, num_subcores=NUM_SUBCORES)


@functools.partial(
    pl.kernel,
    out_type=jax.ShapeDtypeStruct((B, DIM), jnp.float32),
    mesh=_mesh,
    scratch_types=[
        pltpu.VMEM((2 * B_PER_W, HALF), jnp.int32),
        pltpu.VMEM((L, DIM), jnp.float32),
        pltpu.VMEM((L, DIM), jnp.float32),
        pltpu.VMEM((B_PER_W, DIM), jnp.float32),
        pltpu.SemaphoreType.DMA,
        pltpu.SemaphoreType.DMA,
    ],
    compiler_params=pltpu.CompilerParams(use_tc_tiling_on_sc=False),
)
def _embed_mean(idx_hbm, table_hbm, out_hbm, idx_v, rows0, rows1, out_v,
                sem0, sem1):
    wid = lax.axis_index("s") * NUM_CORES + lax.axis_index("c")
    row_base = wid * (2 * B_PER_W)

    pltpu.sync_copy(idx_hbm.at[pl.ds(row_base, 2 * B_PER_W)], idx_v)

    def issue(buf, sem, b):
        pltpu.async_copy(table_hbm.at[idx_v.at[2 * b]],
                         buf.at[pl.ds(0, HALF)], sem)
        pltpu.async_copy(table_hbm.at[idx_v.at[2 * b + 1]],
                         buf.at[pl.ds(HALF, HALF)], sem)

    def wait(buf, sem):
        pltpu.make_async_copy(table_hbm.at[pl.ds(0, L)], buf, sem).wait()

    issue(rows0, sem0, 0)
    issue(rows1, sem1, 1)

    scale = jnp.float32(1.0 / L)

    def accumulate(buf, b):
        def acc_body(l, carry):
            return tuple(carry[c] + buf[l, pl.ds(c * LANES, LANES)]
                         for c in range(NCH))
        zero = jnp.zeros((LANES,), jnp.float32)
        acc = lax.fori_loop(0, L, acc_body, (zero,) * NCH)
        for c in range(NCH):
            out_v[b, pl.ds(c * LANES, LANES)] = acc[c] * scale

    def outer_body(i, carry):
        b0 = 2 * i
        for buf, sem, off in ((rows0, sem0, 0), (rows1, sem1, 1)):
            b = b0 + off
            wait(buf, sem)
            accumulate(buf, b)

            @pl.when(b + 2 < B_PER_W)
            def _():
                issue(buf, sem, b + 2)
        return carry

    lax.fori_loop(0, B_PER_W // 2, outer_body, 0)

    pltpu.sync_copy(out_v, out_hbm.at[pl.ds(wid * B_PER_W, B_PER_W)])


def kernel(indices, table):
    idx2 = indices.reshape(2 * B, HALF)
    return _embed_mean(idx2, table)


# speedup vs baseline: 1.0205x; 1.0205x over previous
import functools

import jax
import jax.numpy as jnp
from jax import lax
from jax.experimental import pallas as pl
from jax.experimental.pallas import tpu as pltpu
from jax.experimental.pallas import tpu_sc as plsc

VOCAB = 1000000
DIM = 64
B = 4096
L = 200

NUM_CORES = 2
NUM_SUBCORES = 16
NW = NUM_CORES * NUM_SUBCORES
B_PER_W = B // NW
HALF = L // 2
LANES = 16
NCH = DIM // LANES

_mesh = plsc.VectorSubcoreMesh(
    core_axis_name="c", subcore_axis_name="s",
    num_cores=NUM_CORES, num_subcores=NUM_SUBCORES)


@functools.partial(
    pl.kernel,
    out_type=jax.ShapeDtypeStruct((B, DIM), jnp.float32),
    mesh=_mesh,
    scratch_types=[
        pltpu.VMEM((2 * B_PER_W, HALF), jnp.int32),
        pltpu.VMEM((L, DIM), jnp.float32),
        pltpu.VMEM((L, DIM), jnp.float32),
        pltpu.VMEM((B_PER_W, DIM), jnp.float32),
        pltpu.SemaphoreType.DMA,
        pltpu.SemaphoreType.DMA,
    ],
    compiler_params=pltpu.CompilerParams(use_tc_tiling_on_sc=False),
)
def _embed_mean(idx_hbm, table_hbm, out_hbm, idx_v, rows0, rows1, out_v,
                sem0, sem1):
    wid = lax.axis_index("s") * NUM_CORES + lax.axis_index("c")
    row_base = wid * (2 * B_PER_W)

    pltpu.sync_copy(idx_hbm.at[pl.ds(row_base, 2 * B_PER_W)], idx_v)

    def issue(buf, sem, b):
        pltpu.async_copy(table_hbm.at[idx_v.at[2 * b]],
                         buf.at[pl.ds(0, HALF)], sem)
        pltpu.async_copy(table_hbm.at[idx_v.at[2 * b + 1]],
                         buf.at[pl.ds(HALF, HALF)], sem)

    def wait(buf, sem):
        pltpu.make_async_copy(table_hbm.at[pl.ds(0, L)], buf, sem).wait()

    issue(rows0, sem0, 0)
    issue(rows1, sem1, 1)

    scale = jnp.float32(1.0 / L)

    UNROLL = 8

    def accumulate(buf, b):
        def acc_body(i, carry):
            acc = list(carry)
            base = i * UNROLL
            for r in range(UNROLL):
                for c in range(NCH):
                    k = c * 2 + (r % 2)
                    acc[k] = acc[k] + buf[base + r, pl.ds(c * LANES, LANES)]
            return tuple(acc)

        zero = jnp.zeros((LANES,), jnp.float32)
        acc = lax.fori_loop(0, L // UNROLL, acc_body, (zero,) * (2 * NCH))
        for c in range(NCH):
            out_v[b, pl.ds(c * LANES, LANES)] = (acc[c * 2] + acc[c * 2 + 1]) * scale

    def outer_body(i, carry):
        b0 = 2 * i
        for buf, sem, off in ((rows0, sem0, 0), (rows1, sem1, 1)):
            b = b0 + off
            wait(buf, sem)
            accumulate(buf, b)

            @pl.when(b + 2 < B_PER_W)
            def _():
                issue(buf, sem, b + 2)
        return carry

    lax.fori_loop(0, B_PER_W // 2, outer_body, 0)

    pltpu.sync_copy(out_v, out_hbm.at[pl.ds(wid * B_PER_W, B_PER_W)])


def kernel(indices, table):
    idx2 = indices.reshape(2 * B, HALF)
    return _embed_mean(idx2, table)


# speedup vs baseline: 1.0752x; 1.0536x over previous
import functools

import jax
import jax.numpy as jnp
from jax import lax
from jax.experimental import pallas as pl
from jax.experimental.pallas import tpu as pltpu
from jax.experimental.pallas import tpu_sc as plsc

VOCAB = 1000000
DIM = 64
B = 4096
L = 200

NUM_CORES = 2
NUM_SUBCORES = 16
NW = NUM_CORES * NUM_SUBCORES
B_PER_W = B // NW
HALF = L // 2
LANES = 16
NCH = DIM // LANES

_mesh = plsc.VectorSubcoreMesh(
    core_axis_name="c", subcore_axis_name="s",
    num_cores=NUM_CORES, num_subcores=NUM_SUBCORES)


@functools.partial(
    pl.kernel,
    out_type=jax.ShapeDtypeStruct((B, DIM), jnp.float32),
    mesh=_mesh,
    scratch_types=[
        pltpu.VMEM((2 * B_PER_W, HALF), jnp.int32),
        pltpu.VMEM((L, DIM), jnp.float32),
        pltpu.VMEM((L, DIM), jnp.float32),
        pltpu.VMEM((L, DIM), jnp.float32),
        pltpu.VMEM((L, DIM), jnp.float32),
        pltpu.VMEM((B_PER_W, DIM), jnp.float32),
        pltpu.SemaphoreType.DMA,
        pltpu.SemaphoreType.DMA,
        pltpu.SemaphoreType.DMA,
        pltpu.SemaphoreType.DMA,
    ],
    compiler_params=pltpu.CompilerParams(use_tc_tiling_on_sc=False),
)
def _embed_mean(idx_hbm, table_hbm, out_hbm, idx_v, rows0, rows1, rows2,
                rows3, out_v, sem0, sem1, sem2, sem3):
    wid = lax.axis_index("s") * NUM_CORES + lax.axis_index("c")
    row_base = wid * (2 * B_PER_W)

    pltpu.sync_copy(idx_hbm.at[pl.ds(row_base, 2 * B_PER_W)], idx_v)

    def issue(buf, sem, b):
        pltpu.async_copy(table_hbm.at[idx_v.at[2 * b]],
                         buf.at[pl.ds(0, HALF)], sem)
        pltpu.async_copy(table_hbm.at[idx_v.at[2 * b + 1]],
                         buf.at[pl.ds(HALF, HALF)], sem)

    def wait(buf, sem):
        pltpu.make_async_copy(table_hbm.at[pl.ds(0, L)], buf, sem).wait()

    ring = ((rows0, sem0, 0), (rows1, sem1, 1),
            (rows2, sem2, 2), (rows3, sem3, 3))
    for buf, sem, off in ring:
        issue(buf, sem, off)

    scale = jnp.float32(1.0 / L)

    UNROLL = 8

    def accumulate(buf, b):
        def acc_body(i, carry):
            acc = list(carry)
            base = i * UNROLL
            for r in range(UNROLL):
                for c in range(NCH):
                    k = c * 2 + (r % 2)
                    acc[k] = acc[k] + buf[base + r, pl.ds(c * LANES, LANES)]
            return tuple(acc)

        zero = jnp.zeros((LANES,), jnp.float32)
        acc = lax.fori_loop(0, L // UNROLL, acc_body, (zero,) * (2 * NCH))
        for c in range(NCH):
            out_v[b, pl.ds(c * LANES, LANES)] = (acc[c * 2] + acc[c * 2 + 1]) * scale

    NBUF = len(ring)

    def outer_body(i, carry):
        b0 = NBUF * i
        for buf, sem, off in ring:
            b = b0 + off
            wait(buf, sem)
            accumulate(buf, b)

            @pl.when(b + NBUF < B_PER_W)
            def _():
                issue(buf, sem, b + NBUF)
        return carry

    lax.fori_loop(0, B_PER_W // NBUF, outer_body, 0)

    pltpu.sync_copy(out_v, out_hbm.at[pl.ds(wid * B_PER_W, B_PER_W)])


def kernel(indices, table):
    idx2 = indices.reshape(2 * B, HALF)
    return _embed_mean(idx2, table)


# speedup vs baseline: 1.0765x; 1.0012x over previous
import functools

import jax
import jax.numpy as jnp
from jax import lax
from jax.experimental import pallas as pl
from jax.experimental.pallas import tpu as pltpu
from jax.experimental.pallas import tpu_sc as plsc

VOCAB = 1000000
DIM = 64
B = 4096
L = 200

NUM_CORES = 2
NUM_SUBCORES = 16
NW = NUM_CORES * NUM_SUBCORES
B_PER_W = B // NW
HALF = L // 2
LANES = 16
NCH = DIM // LANES

_mesh = plsc.VectorSubcoreMesh(
    core_axis_name="c", subcore_axis_name="s",
    num_cores=NUM_CORES, num_subcores=NUM_SUBCORES)


@functools.partial(
    pl.kernel,
    out_type=jax.ShapeDtypeStruct((B, DIM), jnp.float32),
    mesh=_mesh,
    scratch_types=[
        pltpu.VMEM((2 * B_PER_W, HALF), jnp.int32),
        pltpu.VMEM((L, DIM), jnp.float32),
        pltpu.VMEM((L, DIM), jnp.float32),
        pltpu.VMEM((L, DIM), jnp.float32),
        pltpu.VMEM((L, DIM), jnp.float32),
        pltpu.VMEM((L, DIM), jnp.float32),
        pltpu.VMEM((L, DIM), jnp.float32),
        pltpu.VMEM((B_PER_W, DIM), jnp.float32),
        pltpu.SemaphoreType.DMA,
        pltpu.SemaphoreType.DMA,
        pltpu.SemaphoreType.DMA,
        pltpu.SemaphoreType.DMA,
        pltpu.SemaphoreType.DMA,
        pltpu.SemaphoreType.DMA,
    ],
    compiler_params=pltpu.CompilerParams(use_tc_tiling_on_sc=False),
)
def _embed_mean(idx_hbm, table_hbm, out_hbm, idx_v, rows0, rows1, rows2,
                rows3, rows4, rows5, out_v, sem0, sem1, sem2, sem3, sem4,
                sem5):
    wid = lax.axis_index("s") * NUM_CORES + lax.axis_index("c")
    row_base = wid * (2 * B_PER_W)

    pltpu.sync_copy(idx_hbm.at[pl.ds(row_base, 2 * B_PER_W)], idx_v)

    def issue(buf, sem, b):
        pltpu.async_copy(table_hbm.at[idx_v.at[2 * b]],
                         buf.at[pl.ds(0, HALF)], sem)
        pltpu.async_copy(table_hbm.at[idx_v.at[2 * b + 1]],
                         buf.at[pl.ds(HALF, HALF)], sem)

    def wait(buf, sem):
        pltpu.make_async_copy(table_hbm.at[pl.ds(0, L)], buf, sem).wait()

    ring = ((rows0, sem0, 0), (rows1, sem1, 1),
            (rows2, sem2, 2), (rows3, sem3, 3),
            (rows4, sem4, 4), (rows5, sem5, 5))
    for buf, sem, off in ring:
        issue(buf, sem, off)

    scale = jnp.float32(1.0 / L)

    UNROLL = 8

    def accumulate(buf, b):
        def acc_body(i, carry):
            acc = list(carry)
            base = i * UNROLL
            for r in range(UNROLL):
                for c in range(NCH):
                    k = c * 2 + (r % 2)
                    acc[k] = acc[k] + buf[base + r, pl.ds(c * LANES, LANES)]
            return tuple(acc)

        zero = jnp.zeros((LANES,), jnp.float32)
        acc = lax.fori_loop(0, L // UNROLL, acc_body, (zero,) * (2 * NCH))
        for c in range(NCH):
            out_v[b, pl.ds(c * LANES, LANES)] = (acc[c * 2] + acc[c * 2 + 1]) * scale

    NBUF = len(ring)

    def outer_body(i, carry):
        b0 = NBUF * i
        for buf, sem, off in ring:
            b = b0 + off
            wait(buf, sem)
            accumulate(buf, b)

            @pl.when(b + NBUF < B_PER_W)
            def _():
                issue(buf, sem, b + NBUF)
        return carry

    lax.fori_loop(0, B_PER_W // NBUF, outer_body, 0)

    REM = B_PER_W % NBUF
    for r in range(REM):
        buf, sem, _ = ring[r]
        wait(buf, sem)
        accumulate(buf, B_PER_W - REM + r)

    pltpu.sync_copy(out_v, out_hbm.at[pl.ds(wid * B_PER_W, B_PER_W)])


def kernel(indices, table):
    idx2 = indices.reshape(2 * B, HALF)
    return _embed_mean(idx2, table)


# speedup vs baseline: 1.0783x; 1.0017x over previous
import functools

import jax
import jax.numpy as jnp
from jax import lax
from jax.experimental import pallas as pl
from jax.experimental.pallas import tpu as pltpu
from jax.experimental.pallas import tpu_sc as plsc

VOCAB = 1000000
DIM = 64
B = 4096
L = 200

NUM_CORES = 2
NUM_SUBCORES = 16
NW = NUM_CORES * NUM_SUBCORES
B_PER_W = B // NW
IDX_PER_W = B_PER_W * L
BPD = 2
DROWS = BPD * L
NDESC = B_PER_W // BPD
LANES = 16
NCH = DIM // LANES

_mesh = plsc.VectorSubcoreMesh(
    core_axis_name="c", subcore_axis_name="s",
    num_cores=NUM_CORES, num_subcores=NUM_SUBCORES)


@functools.partial(
    pl.kernel,
    out_type=jax.ShapeDtypeStruct((B, DIM), jnp.float32),
    mesh=_mesh,
    scratch_types=[
        pltpu.VMEM((IDX_PER_W,), jnp.int32),
        pltpu.VMEM((DROWS, DIM), jnp.float32),
        pltpu.VMEM((DROWS, DIM), jnp.float32),
        pltpu.VMEM((DROWS, DIM), jnp.float32),
        pltpu.VMEM((B_PER_W, DIM), jnp.float32),
        pltpu.SemaphoreType.DMA,
        pltpu.SemaphoreType.DMA,
        pltpu.SemaphoreType.DMA,
    ],
    compiler_params=pltpu.CompilerParams(use_tc_tiling_on_sc=False),
)
def _embed_mean(idx_hbm, table_hbm, out_hbm, idx_v, rows0, rows1, rows2,
                out_v, sem0, sem1, sem2):
    wid = lax.axis_index("s") * NUM_CORES + lax.axis_index("c")

    pltpu.sync_copy(idx_hbm.at[pl.ds(wid * IDX_PER_W, IDX_PER_W)], idx_v)

    def issue(buf, sem, g):
        pltpu.async_copy(table_hbm.at[idx_v.at[pl.ds(g * DROWS, DROWS)]],
                         buf, sem)

    def wait(buf, sem):
        pltpu.make_async_copy(table_hbm.at[pl.ds(0, DROWS)], buf, sem).wait()

    ring = ((rows0, sem0, 0), (rows1, sem1, 1), (rows2, sem2, 2))
    NBUF = len(ring)
    for buf, sem, off in ring:
        issue(buf, sem, off)

    scale = jnp.float32(1.0 / L)
    UNROLL = 8

    def accumulate(buf, g):
        for j in range(BPD):
            b = g * BPD + j

            def acc_body(i, carry, j=j):
                acc = list(carry)
                base = j * L + i * UNROLL
                for r in range(UNROLL):
                    for c in range(NCH):
                        k = c * 2 + (r % 2)
                        acc[k] = acc[k] + buf[base + r,
                                              pl.ds(c * LANES, LANES)]
                return tuple(acc)

            zero = jnp.zeros((LANES,), jnp.float32)
            acc = lax.fori_loop(0, L // UNROLL, acc_body, (zero,) * (2 * NCH))
            for c in range(NCH):
                out_v[b, pl.ds(c * LANES, LANES)] = (
                    acc[c * 2] + acc[c * 2 + 1]) * scale

    def outer_body(i, carry):
        g0 = NBUF * i
        for buf, sem, off in ring:
            g = g0 + off
            wait(buf, sem)
            accumulate(buf, g)

            @pl.when(g + NBUF < NDESC)
            def _():
                issue(buf, sem, g + NBUF)
        return carry

    lax.fori_loop(0, NDESC // NBUF, outer_body, 0)

    REM = NDESC % NBUF
    for r in range(REM):
        buf, sem, _ = ring[r]
        wait(buf, sem)
        accumulate(buf, NDESC - REM + r)

    pltpu.sync_copy(out_v, out_hbm.at[pl.ds(wid * B_PER_W, B_PER_W)])


def kernel(indices, table):
    return _embed_mean(indices.reshape(-1), table)
